# Initial kernel scaffold; baseline (speedup 1.0000x reference)
import jax
import jax.numpy as jnp
from jax.experimental import pallas as pl


def kernel(X, triplet, W1, b1, W2, b2):
    raise NotImplementedError("write your pallas kernel here")


# speedup vs baseline: 284.2471x; 284.2471x over previous
import jax
import jax.numpy as jnp
from jax.experimental import pallas as pl
from jax.experimental.pallas import tpu as pltpu

_BLK = 1000


def _pass_a(x_ref, h_ref, s1t_ref, bc_ref):
    @pl.when(pl.program_id(0) == 0)
    def _():
        s1t_ref[...] = jnp.zeros_like(s1t_ref)
        bc_ref[...] = jnp.zeros_like(bc_ref)

    hb = h_ref[...]
    s1t_ref[...] += jax.lax.dot_general(
        x_ref[...], hb, (((0,), (0,)), ((), ())),
        preferred_element_type=jnp.float32)
    bc_ref[...] += jnp.sum(hb, axis=0, keepdims=True)


def _pass_b(h_ref, s1t_ref, bc_ref, w1_ref, b1_ref, s2t_ref, oe_ref):
    @pl.when(pl.program_id(0) == 0)
    def _():
        bc = bc_ref[...]
        binv = jnp.where(bc > 0, 1.0 / bc, 0.0)
        oe_ref[...] = binv * jnp.dot(
            w1_ref[...], s1t_ref[...],
            preferred_element_type=jnp.float32)
        s2t_ref[...] = jnp.zeros_like(s2t_ref)

    hb = h_ref[...]
    d = jnp.sum(hb, axis=1, keepdims=True)
    dinv = jnp.where(d > 0, 1.0 / d, 0.0)
    y = jax.lax.dot_general(
        hb, oe_ref[...], (((1,), (1,)), ((), ())),
        preferred_element_type=jnp.float32)
    hfeat = jnp.maximum(dinv * y + b1_ref[...], 0.0)
    s2t_ref[...] += jax.lax.dot_general(
        hfeat, hb, (((0,), (0,)), ((), ())),
        preferred_element_type=jnp.float32)


def _pass_c(h_ref, s2t_ref, bc_ref, w2_ref, b2_ref, out_ref, oe_ref):
    @pl.when(pl.program_id(0) == 0)
    def _():
        bc = bc_ref[...]
        binv = jnp.where(bc > 0, 1.0 / bc, 0.0)
        oe_ref[...] = binv * jnp.dot(
            w2_ref[...], s2t_ref[...],
            preferred_element_type=jnp.float32)

    hb = h_ref[...]
    d = jnp.sum(hb, axis=1, keepdims=True)
    dinv = jnp.where(d > 0, 1.0 / d, 0.0)
    y = jax.lax.dot_general(
        hb, oe_ref[...], (((1,), (1,)), ((), ())),
        preferred_element_type=jnp.float32)
    out_ref[...] = dinv * y + b2_ref[...]


def kernel(X, triplet, W1, b1, W2, b2):
    N, Din = X.shape
    E = triplet.shape[0]
    Dh = W1.shape[0]
    Dout = W2.shape[0]
    nb = N // _BLK

    Hf = (triplet != 0).T.astype(jnp.float32)
    b1r = b1.reshape(1, Dh)
    b2r = b2.reshape(1, Dout)

    s1t, bc = pl.pallas_call(
        _pass_a,
        grid=(nb,),
        in_specs=[
            pl.BlockSpec((_BLK, Din), lambda i: (i, 0)),
            pl.BlockSpec((_BLK, E), lambda i: (i, 0)),
        ],
        out_specs=[
            pl.BlockSpec((Din, E), lambda i: (0, 0)),
            pl.BlockSpec((1, E), lambda i: (0, 0)),
        ],
        out_shape=[
            jax.ShapeDtypeStruct((Din, E), jnp.float32),
            jax.ShapeDtypeStruct((1, E), jnp.float32),
        ],
    )(X, Hf)

    s2t = pl.pallas_call(
        _pass_b,
        grid=(nb,),
        in_specs=[
            pl.BlockSpec((_BLK, E), lambda i: (i, 0)),
            pl.BlockSpec((Din, E), lambda i: (0, 0)),
            pl.BlockSpec((1, E), lambda i: (0, 0)),
            pl.BlockSpec((Dh, Din), lambda i: (0, 0)),
            pl.BlockSpec((1, Dh), lambda i: (0, 0)),
        ],
        out_specs=pl.BlockSpec((Dh, E), lambda i: (0, 0)),
        out_shape=jax.ShapeDtypeStruct((Dh, E), jnp.float32),
        scratch_shapes=[pltpu.VMEM((Dh, E), jnp.float32)],
    )(Hf, s1t, bc, W1, b1r)

    out = pl.pallas_call(
        _pass_c,
        grid=(nb,),
        in_specs=[
            pl.BlockSpec((_BLK, E), lambda i: (i, 0)),
            pl.BlockSpec((Dh, E), lambda i: (0, 0)),
            pl.BlockSpec((1, E), lambda i: (0, 0)),
            pl.BlockSpec((Dout, Dh), lambda i: (0, 0)),
            pl.BlockSpec((1, Dout), lambda i: (0, 0)),
        ],
        out_specs=pl.BlockSpec((_BLK, Dout), lambda i: (i, 0)),
        out_shape=jax.ShapeDtypeStruct((N, Dout), jnp.float32),
        scratch_shapes=[pltpu.VMEM((Dout, E), jnp.float32)],
    )(Hf, s2t, bc, W2, b2r)

    return out


# speedup vs baseline: 291.2599x; 1.0247x over previous
import jax
import jax.numpy as jnp
from jax.experimental import pallas as pl
from jax.experimental.pallas import tpu as pltpu

_BLK = 1000


def _fused(x_ref, h_ref, w1_ref, b1_ref, w2_ref, b2_ref, out_ref,
           s1t_ref, bc_ref, s2t_ref, oe_ref):
    s = pl.program_id(0)
    i = pl.program_id(1)
    hb = h_ref[pl.ds(i * _BLK, _BLK), :]

    @pl.when(jnp.logical_and(s == 0, i == 0))
    def _():
        s1t_ref[...] = jnp.zeros_like(s1t_ref)
        bc_ref[...] = jnp.zeros_like(bc_ref)

    @pl.when(s == 0)
    def _():
        s1t_ref[...] += jax.lax.dot_general(
            x_ref[...], hb, (((0,), (0,)), ((), ())),
            preferred_element_type=jnp.float32)
        bc_ref[...] += jnp.sum(hb, axis=0, keepdims=True)

    @pl.when(jnp.logical_and(s == 1, i == 0))
    def _():
        bc = bc_ref[...]
        binv = jnp.where(bc > 0, 1.0 / bc, 0.0)
        oe_ref[...] = binv * jnp.dot(
            w1_ref[...], s1t_ref[...],
            preferred_element_type=jnp.float32)
        s2t_ref[...] = jnp.zeros_like(s2t_ref)

    d = jnp.sum(hb, axis=1, keepdims=True)
    dinv = jnp.where(d > 0, 1.0 / d, 0.0)

    @pl.when(s == 1)
    def _():
        y = jax.lax.dot_general(
            hb, oe_ref[...], (((1,), (1,)), ((), ())),
            preferred_element_type=jnp.float32)
        hfeat = jnp.maximum(dinv * y + b1_ref[...], 0.0)
        s2t_ref[...] += jax.lax.dot_general(
            hfeat, hb, (((0,), (0,)), ((), ())),
            preferred_element_type=jnp.float32)

    @pl.when(jnp.logical_and(s == 2, i == 0))
    def _():
        bc = bc_ref[...]
        binv = jnp.where(bc > 0, 1.0 / bc, 0.0)
        oe_ref[...] = binv * jnp.dot(
            w2_ref[...], s2t_ref[...],
            preferred_element_type=jnp.float32)

    @pl.when(s == 2)
    def _():
        y = jax.lax.dot_general(
            hb, oe_ref[...], (((1,), (1,)), ((), ())),
            preferred_element_type=jnp.float32)
        out_ref[...] = dinv * y + b2_ref[...]


def kernel(X, triplet, W1, b1, W2, b2):
    N, Din = X.shape
    E = triplet.shape[0]
    Dh = W1.shape[0]
    Dout = W2.shape[0]
    nb = N // _BLK

    Hf = (triplet != 0).T.astype(jnp.float32)
    b1r = b1.reshape(1, Dh)
    b2r = b2.reshape(1, Dout)

    out = pl.pallas_call(
        _fused,
        grid=(3, nb),
        in_specs=[
            pl.BlockSpec((_BLK, Din),
                         lambda s, i: (jnp.where(s == 0, i, 0), 0)),
            pl.BlockSpec((N, E), lambda s, i: (0, 0)),
            pl.BlockSpec((Dh, Din), lambda s, i: (0, 0)),
            pl.BlockSpec((1, Dh), lambda s, i: (0, 0)),
            pl.BlockSpec((Dout, Dh), lambda s, i: (0, 0)),
            pl.BlockSpec((1, Dout), lambda s, i: (0, 0)),
        ],
        out_specs=pl.BlockSpec((_BLK, Dout),
                               lambda s, i: (jnp.where(s == 2, i, 0), 0)),
        out_shape=jax.ShapeDtypeStruct((N, Dout), jnp.float32),
        scratch_shapes=[
            pltpu.VMEM((Din, E), jnp.float32),
            pltpu.VMEM((1, E), jnp.float32),
            pltpu.VMEM((Dh, E), jnp.float32),
            pltpu.VMEM((max(Dh, Dout), E), jnp.float32),
        ],
    )(X, Hf, W1, b1r, W2, b2r)

    return out


# speedup vs baseline: 366.4834x; 1.2583x over previous
import jax
import jax.numpy as jnp
from jax.experimental import pallas as pl
from jax.experimental.pallas import tpu as pltpu

_BLK = 2000


def _fused(x_ref, h_ref, w1_ref, b1_ref, w2_ref, b2_ref, out_ref,
           s1t_ref, bc_ref, s2t_ref, oe_ref):
    s = pl.program_id(0)
    i = pl.program_id(1)
    hb = h_ref[pl.ds(i * _BLK, _BLK), :]

    @pl.when(jnp.logical_and(s == 0, i == 0))
    def _():
        s1t_ref[...] = jnp.zeros_like(s1t_ref)
        bc_ref[...] = jnp.zeros_like(bc_ref)

    @pl.when(s == 0)
    def _():
        s1t_ref[...] += jax.lax.dot_general(
            x_ref[...], hb, (((0,), (0,)), ((), ())),
            preferred_element_type=jnp.float32)
        bc_ref[...] += jnp.sum(hb, axis=0, keepdims=True)

    @pl.when(jnp.logical_and(s == 1, i == 0))
    def _():
        bc = bc_ref[...]
        binv = jnp.where(bc > 0, 1.0 / bc, 0.0)
        oe_ref[...] = binv * jnp.dot(
            w1_ref[...], s1t_ref[...],
            preferred_element_type=jnp.float32)
        s2t_ref[...] = jnp.zeros_like(s2t_ref)

    d = jnp.sum(hb, axis=1, keepdims=True)
    dinv = jnp.where(d > 0, 1.0 / d, 0.0)

    @pl.when(s == 1)
    def _():
        y = jax.lax.dot_general(
            hb, oe_ref[...], (((1,), (1,)), ((), ())),
            preferred_element_type=jnp.float32)
        hfeat = jnp.maximum(dinv * y + b1_ref[...], 0.0)
        s2t_ref[...] += jax.lax.dot_general(
            hfeat, hb, (((0,), (0,)), ((), ())),
            preferred_element_type=jnp.float32)

    @pl.when(jnp.logical_and(s == 2, i == 0))
    def _():
        bc = bc_ref[...]
        binv = jnp.where(bc > 0, 1.0 / bc, 0.0)
        oe_ref[...] = binv * jnp.dot(
            w2_ref[...], s2t_ref[...],
            preferred_element_type=jnp.float32)

    @pl.when(s == 2)
    def _():
        y = jax.lax.dot_general(
            hb, oe_ref[...], (((1,), (1,)), ((), ())),
            preferred_element_type=jnp.float32)
        out_ref[...] = dinv * y + b2_ref[...]


def kernel(X, triplet, W1, b1, W2, b2):
    N, Din = X.shape
    E = triplet.shape[0]
    Dh = W1.shape[0]
    Dout = W2.shape[0]
    nb = N // _BLK

    Hf = (triplet != 0).T.astype(jnp.float32)
    b1r = b1.reshape(1, Dh)
    b2r = b2.reshape(1, Dout)

    out = pl.pallas_call(
        _fused,
        grid=(3, nb),
        in_specs=[
            pl.BlockSpec((_BLK, Din),
                         lambda s, i: (jnp.where(s == 0, i, 0), 0)),
            pl.BlockSpec((N, E), lambda s, i: (0, 0)),
            pl.BlockSpec((Dh, Din), lambda s, i: (0, 0)),
            pl.BlockSpec((1, Dh), lambda s, i: (0, 0)),
            pl.BlockSpec((Dout, Dh), lambda s, i: (0, 0)),
            pl.BlockSpec((1, Dout), lambda s, i: (0, 0)),
        ],
        out_specs=pl.BlockSpec((_BLK, Dout),
                               lambda s, i: (jnp.where(s == 2, i, 0), 0)),
        out_shape=jax.ShapeDtypeStruct((N, Dout), jnp.float32),
        scratch_shapes=[
            pltpu.VMEM((Din, E), jnp.float32),
            pltpu.VMEM((1, E), jnp.float32),
            pltpu.VMEM((Dh, E), jnp.float32),
            pltpu.VMEM((max(Dh, Dout), E), jnp.float32),
        ],
    )(X, Hf, W1, b1r, W2, b2r)

    return out


# speedup vs baseline: 408.8824x; 1.1157x over previous
import jax
import jax.numpy as jnp
from jax.experimental import pallas as pl
from jax.experimental.pallas import tpu as pltpu

_BLK = 5000


def _fused(x_ref, h_ref, w1_ref, b1_ref, w2_ref, b2_ref, out_ref,
           s1t_ref, bc_ref, s2t_ref, oe_ref):
    s = pl.program_id(0)
    i = pl.program_id(1)
    hb = h_ref[pl.ds(i * _BLK, _BLK), :]

    @pl.when(jnp.logical_and(s == 0, i == 0))
    def _():
        s1t_ref[...] = jnp.zeros_like(s1t_ref)
        bc_ref[...] = jnp.zeros_like(bc_ref)

    @pl.when(s == 0)
    def _():
        s1t_ref[...] += jax.lax.dot_general(
            x_ref[...], hb, (((0,), (0,)), ((), ())),
            preferred_element_type=jnp.float32)
        bc_ref[...] += jnp.sum(hb, axis=0, keepdims=True)

    @pl.when(jnp.logical_and(s == 1, i == 0))
    def _():
        bc = bc_ref[...]
        binv = jnp.where(bc > 0, 1.0 / bc, 0.0)
        oe_ref[...] = binv * jnp.dot(
            w1_ref[...], s1t_ref[...],
            preferred_element_type=jnp.float32)
        s2t_ref[...] = jnp.zeros_like(s2t_ref)

    d = jnp.sum(hb, axis=1, keepdims=True)
    dinv = jnp.where(d > 0, 1.0 / d, 0.0)

    @pl.when(s == 1)
    def _():
        y = jax.lax.dot_general(
            hb, oe_ref[...], (((1,), (1,)), ((), ())),
            preferred_element_type=jnp.float32)
        hfeat = jnp.maximum(dinv * y + b1_ref[...], 0.0)
        s2t_ref[...] += jax.lax.dot_general(
            hfeat, hb, (((0,), (0,)), ((), ())),
            preferred_element_type=jnp.float32)

    @pl.when(jnp.logical_and(s == 2, i == 0))
    def _():
        bc = bc_ref[...]
        binv = jnp.where(bc > 0, 1.0 / bc, 0.0)
        oe_ref[...] = binv * jnp.dot(
            w2_ref[...], s2t_ref[...],
            preferred_element_type=jnp.float32)

    @pl.when(s == 2)
    def _():
        y = jax.lax.dot_general(
            hb, oe_ref[...], (((1,), (1,)), ((), ())),
            preferred_element_type=jnp.float32)
        out_ref[...] = dinv * y + b2_ref[...]


def kernel(X, triplet, W1, b1, W2, b2):
    N, Din = X.shape
    E = triplet.shape[0]
    Dh = W1.shape[0]
    Dout = W2.shape[0]
    nb = N // _BLK

    Hf = (triplet != 0).T.astype(jnp.float32)
    b1r = b1.reshape(1, Dh)
    b2r = b2.reshape(1, Dout)

    out = pl.pallas_call(
        _fused,
        grid=(3, nb),
        in_specs=[
            pl.BlockSpec((_BLK, Din),
                         lambda s, i: (jnp.where(s == 0, i, 0), 0)),
            pl.BlockSpec((N, E), lambda s, i: (0, 0)),
            pl.BlockSpec((Dh, Din), lambda s, i: (0, 0)),
            pl.BlockSpec((1, Dh), lambda s, i: (0, 0)),
            pl.BlockSpec((Dout, Dh), lambda s, i: (0, 0)),
            pl.BlockSpec((1, Dout), lambda s, i: (0, 0)),
        ],
        out_specs=pl.BlockSpec((_BLK, Dout),
                               lambda s, i: (jnp.where(s == 2, i, 0), 0)),
        out_shape=jax.ShapeDtypeStruct((N, Dout), jnp.float32),
        scratch_shapes=[
            pltpu.VMEM((Din, E), jnp.float32),
            pltpu.VMEM((1, E), jnp.float32),
            pltpu.VMEM((Dh, E), jnp.float32),
            pltpu.VMEM((max(Dh, Dout), E), jnp.float32),
        ],
    )(X, Hf, W1, b1r, W2, b2r)

    return out
